# baseline (speedup 1.0000x reference)
import jax
import jax.numpy as jnp
from jax import lax
from jax.experimental import pallas as pl
from jax.experimental.pallas import tpu as pltpu
from jax.experimental.pallas import tpu_sc as plsc

NC = 2
NS = 16
NW = NC * NS

V = 1000000
D = 32
NBLK = (V + 127) // 128
VP = NBLK * 128

N_TOK = 16384
S_TOK = 50
B = N_TOK * S_TOK
NOUT = B // 128
K1 = 8


TB = 8
B0_FULL = (NBLK // NW) // TB


def _b0_body(lut_t, tail, table, in_v, out_v):
    wid = lax.axis_index("s") * NC + lax.axis_index("c")
    n_c = NBLK // NW + jnp.where(wid < NBLK % NW, 1, 0)
    start = wid * (NBLK // NW) + jnp.minimum(wid, NBLK % NW)

    iota16 = lax.broadcasted_iota(jnp.int32, (16,), 0)
    idx_f = [j0 % 32 + iota16 for j0 in (0, 16)]

    def permute_tile(kk, lane0, row0):
        def per_i(i2, carry):
            vals = []
            for half in range(2):
                i = 2 * i2 + half
                for u in range(4):
                    idx_l = jnp.full((16,), lane0 + 4 * i + u, jnp.int32)
                    for par in range(2):
                        vals.append(plsc.load_gather(in_v, [idx_f[par], idx_l]))
            for t8, v in enumerate(vals):
                out_v[row0 + 2 * i2 + t8 // 8, pl.ds(16 * (t8 % 8), 16)] = v
            return carry
        lax.fori_loop(0, 16, per_i, 0)

    def per_batch(g, carry):
        c0 = start + g * TB
        pltpu.sync_copy(lut_t.at[:, pl.ds(c0 * 128, TB * 128)], in_v)

        def per_k(kk, carry2):
            permute_tile(kk, kk * 128, kk * 32)
            return carry2
        lax.fori_loop(0, TB, per_k, 0)
        pltpu.sync_copy(out_v, table.at[pl.ds(c0 * 32, TB * 32)])
        return carry

    lax.fori_loop(0, B0_FULL, per_batch, 0)

    def per_tile(t, carry):
        c = start + B0_FULL * TB + t
        is_last = c == NBLK - 1

        @pl.when(jnp.logical_not(is_last))
        def _():
            pltpu.sync_copy(lut_t.at[:, pl.ds(c * 128, 128)], in_v.at[:, pl.ds(0, 128)])

        @pl.when(is_last)
        def _():
            pltpu.sync_copy(tail, in_v.at[:, pl.ds(0, 128)])

        permute_tile(0, 0, 0)
        pltpu.sync_copy(out_v.at[pl.ds(0, 32)], table.at[pl.ds(c * 32, 32)])
        return carry

    lax.fori_loop(0, n_c - B0_FULL * TB, per_tile, 0)


def _relayout(lut):
    tail = jnp.pad(lut[(NBLK - 1) * 128:], ((0, VP - V), (0, 0))).T
    mesh = plsc.VectorSubcoreMesh(core_axis_name="c", subcore_axis_name="s")
    k = pl.kernel(
        _b0_body,
        mesh=mesh,
        out_type=jax.ShapeDtypeStruct((NBLK * 32, 128), jnp.float32),
        compiler_params=pltpu.CompilerParams(needs_layout_passes=False),
        scratch_types=[
            pltpu.VMEM((32, TB * 128), jnp.float32),
            pltpu.VMEM((TB * 32, 128), jnp.float32),
        ],
    )
    return k(lut.T, tail)


def _b1_body(idx_hbm, table, out5, idx_v, rows_v, trans_v, sem_g, sem_w):
    wid = lax.axis_index("s") * NC + lax.axis_index("c")
    blocks_per_w = NOUT // NW
    b0 = wid * blocks_per_w

    iota16 = lax.broadcasted_iota(jnp.int32, (16,), 0)

    def per_batch(g, carry):
        base_blk = b0 + g * K1
        pltpu.sync_copy(idx_hbm.at[pl.ds(base_blk * 128, K1 * 128)], idx_v)
        copies = [
            pltpu.async_copy(
                table.at[idx_v.at[pl.ds(k * 128, 128)]],
                rows_v.at[pl.ds(k * 128, 128)],
                sem_g,
            )
            for k in range(K1)
        ]
        for cp in copies:
            cp.wait()

        idx_f = [jnp.full((16,), f, jnp.int32) for f in range(D)]

        def per_block(k, carry2):
            blk = base_blk + k
            s = blk // 128
            nb = blk % 128
            for t8 in range(8):
                n0 = 16 * t8
                idx_n = k * 128 + n0 + iota16
                vals = [
                    plsc.load_gather(rows_v, [idx_n, idx_f[f]])
                    for f in range(D)
                ]
                for f, v in enumerate(vals):
                    trans_v[k, f // 8, f % 8, pl.ds(n0, 16)] = v
            pltpu.async_copy(trans_v.at[k], out5.at[s, :, nb], sem_w)
            return carry2

        lax.fori_loop(0, K1, per_block, 0)
        for _ in range(K1):
            pltpu.make_async_copy(trans_v.at[0], out5.at[0, :, 0], sem_w).wait()
        return carry

    lax.fori_loop(0, blocks_per_w // K1, per_batch, 0)


def _lookup(idx_flat, table):
    mesh = plsc.VectorSubcoreMesh(core_axis_name="c", subcore_axis_name="s")
    k = pl.kernel(
        _b1_body,
        mesh=mesh,
        out_type=jax.ShapeDtypeStruct((S_TOK, 4, 128, 8, 128), jnp.float32),
        compiler_params=pltpu.CompilerParams(
            use_tc_tiling_on_sc=False, needs_layout_passes=False
        ),
        scratch_types=[
            pltpu.VMEM((K1 * 128,), jnp.int32),
            pltpu.VMEM((K1 * 128, D), jnp.float32),
            pltpu.VMEM((K1, 4, 8, 128), jnp.float32),
            pltpu.SemaphoreType.DMA,
            pltpu.SemaphoreType.DMA,
        ],
    )
    return k(idx_flat, table)


def kernel(token_ids, lut):
    idx_bt = token_ids.T.reshape(B).astype(jnp.int32)
    lin = _relayout(lut)
    table = lin.reshape(VP, D)
    out5 = _lookup(idx_bt, table)
    return jnp.transpose(out5, (2, 4, 0, 1, 3)).reshape(N_TOK, S_TOK, D)


# speedup vs baseline: 1.5786x; 1.5786x over previous
import jax
import jax.numpy as jnp
from jax import lax
from jax.experimental import pallas as pl
from jax.experimental.pallas import tpu as pltpu
from jax.experimental.pallas import tpu_sc as plsc

NC = 2
NS = 16
NW = NC * NS

V = 1000000
D = 32
NBLK = (V + 127) // 128
VP = NBLK * 128

N_TOK = 16384
S_TOK = 50
B = N_TOK * S_TOK
NOUT = B // 128
K1 = 8


TB = 8
B0_FULL = (NBLK // NW) // TB


def _b0_body(lut_t, tail, table, in_v, out_v):
    wid = lax.axis_index("s") * NC + lax.axis_index("c")
    n_c = NBLK // NW + jnp.where(wid < NBLK % NW, 1, 0)
    start = wid * (NBLK // NW) + jnp.minimum(wid, NBLK % NW)

    iota16 = lax.broadcasted_iota(jnp.int32, (16,), 0)
    zq = [(iota16 + q) & 15 for q in range(16)]

    def permute_tile(kk, lane0, row0):
        iota_f = [iota16, iota16 + 16]

        def per_sub(e8, carry):
            e0 = e8 * 16
            gbase = jnp.full((16,), lane0 + e0, jnp.int32)
            rbase = jnp.full((16,), row0 + e0 // 4, jnp.int32)
            for f0h in range(2):
                vals = [
                    plsc.load_gather(in_v, [iota_f[f0h], gbase + zq[q]])
                    for q in range(16)
                ]
                for q in range(16):
                    t = zq[q]
                    row_idx = rbase + (t >> 2)
                    lane_idx = ((t & 3) << 5) + iota_f[f0h]
                    plsc.store_scatter(out_v, [row_idx, lane_idx], vals[q])
            return carry

        lax.fori_loop(0, 8, per_sub, 0)

    def per_batch(g, carry):
        c0 = start + g * TB
        pltpu.sync_copy(lut_t.at[:, pl.ds(c0 * 128, TB * 128)], in_v)

        def per_k(kk, carry2):
            permute_tile(kk, kk * 128, kk * 32)
            return carry2
        lax.fori_loop(0, TB, per_k, 0)
        pltpu.sync_copy(out_v, table.at[pl.ds(c0 * 32, TB * 32)])
        return carry

    lax.fori_loop(0, B0_FULL, per_batch, 0)

    def per_tile(t, carry):
        c = start + B0_FULL * TB + t
        is_last = c == NBLK - 1

        @pl.when(jnp.logical_not(is_last))
        def _():
            pltpu.sync_copy(lut_t.at[:, pl.ds(c * 128, 128)], in_v.at[:, pl.ds(0, 128)])

        @pl.when(is_last)
        def _():
            pltpu.sync_copy(tail, in_v.at[:, pl.ds(0, 128)])

        permute_tile(0, 0, 0)
        pltpu.sync_copy(out_v.at[pl.ds(0, 32)], table.at[pl.ds(c * 32, 32)])
        return carry

    lax.fori_loop(0, n_c - B0_FULL * TB, per_tile, 0)


def _relayout(lut):
    tail = jnp.pad(lut[(NBLK - 1) * 128:], ((0, VP - V), (0, 0))).T
    mesh = plsc.VectorSubcoreMesh(core_axis_name="c", subcore_axis_name="s")
    k = pl.kernel(
        _b0_body,
        mesh=mesh,
        out_type=jax.ShapeDtypeStruct((NBLK * 32, 128), jnp.float32),
        compiler_params=pltpu.CompilerParams(needs_layout_passes=False),
        scratch_types=[
            pltpu.VMEM((32, TB * 128), jnp.float32),
            pltpu.VMEM((TB * 32, 128), jnp.float32),
        ],
    )
    return k(lut.T, tail)


def _b1_body(idx_hbm, table, out5, idx_v, rows_v, trans_v, sem_g, sem_w):
    wid = lax.axis_index("s") * NC + lax.axis_index("c")
    blocks_per_w = NOUT // NW
    b0 = wid * blocks_per_w

    iota16 = lax.broadcasted_iota(jnp.int32, (16,), 0)

    def per_batch(g, carry):
        base_blk = b0 + g * K1
        pltpu.sync_copy(idx_hbm.at[pl.ds(base_blk * 128, K1 * 128)], idx_v)
        copies = [
            pltpu.async_copy(
                table.at[idx_v.at[pl.ds(k * 128, 128)]],
                rows_v.at[pl.ds(k * 128, 128)],
                sem_g,
            )
            for k in range(K1)
        ]
        for cp in copies:
            cp.wait()

        idx_f = [jnp.full((16,), f, jnp.int32) for f in range(D)]

        def per_block(k, carry2):
            blk = base_blk + k
            s = blk // 128
            nb = blk % 128
            for t8 in range(8):
                n0 = 16 * t8
                idx_n = k * 128 + n0 + iota16
                vals = [
                    plsc.load_gather(rows_v, [idx_n, idx_f[f]])
                    for f in range(D)
                ]
                for f, v in enumerate(vals):
                    trans_v[k, f // 8, f % 8, pl.ds(n0, 16)] = v
            pltpu.async_copy(trans_v.at[k], out5.at[s, :, nb], sem_w)
            return carry2

        lax.fori_loop(0, K1, per_block, 0)
        for _ in range(K1):
            pltpu.make_async_copy(trans_v.at[0], out5.at[0, :, 0], sem_w).wait()
        return carry

    lax.fori_loop(0, blocks_per_w // K1, per_batch, 0)


def _lookup(idx_flat, table):
    mesh = plsc.VectorSubcoreMesh(core_axis_name="c", subcore_axis_name="s")
    k = pl.kernel(
        _b1_body,
        mesh=mesh,
        out_type=jax.ShapeDtypeStruct((S_TOK, 4, 128, 8, 128), jnp.float32),
        compiler_params=pltpu.CompilerParams(
            use_tc_tiling_on_sc=False, needs_layout_passes=False
        ),
        scratch_types=[
            pltpu.VMEM((K1 * 128,), jnp.int32),
            pltpu.VMEM((K1 * 128, D), jnp.float32),
            pltpu.VMEM((K1, 4, 8, 128), jnp.float32),
            pltpu.SemaphoreType.DMA,
            pltpu.SemaphoreType.DMA,
        ],
    )
    return k(idx_flat, table)


def kernel(token_ids, lut):
    idx_bt = token_ids.T.reshape(B).astype(jnp.int32)
    lin = _relayout(lut)
    table = lin.reshape(VP, D)
    out5 = _lookup(idx_bt, table)
    return jnp.transpose(out5, (2, 4, 0, 1, 3)).reshape(N_TOK, S_TOK, D)


# speedup vs baseline: 2.4085x; 1.5257x over previous
import jax
import jax.numpy as jnp
from jax import lax
from jax.experimental import pallas as pl
from jax.experimental.pallas import tpu as pltpu
from jax.experimental.pallas import tpu_sc as plsc

NC = 2
NS = 16
NW = NC * NS

V = 1000000
D = 32
NBLK = (V + 127) // 128
VP = NBLK * 128

N_TOK = 16384
S_TOK = 50
B = N_TOK * S_TOK
NOUT = B // 128
K1 = 8


TB = 8
B0_FULL = (NBLK // NW) // TB


def _b0_body(lut_t, tail, table, in_v, out_v):
    wid = lax.axis_index("s") * NC + lax.axis_index("c")
    n_c = NBLK // NW + jnp.where(wid < NBLK % NW, 1, 0)
    start = wid * (NBLK // NW) + jnp.minimum(wid, NBLK % NW)

    iota16 = lax.broadcasted_iota(jnp.int32, (16,), 0)
    zq = [(iota16 + q) & 15 for q in range(16)]

    def permute_tile(kk, lane0, row0):
        iota_f = [iota16, iota16 + 16]

        def per_sub(e8, carry):
            e0 = e8 * 16
            gbase = jnp.full((16,), lane0 + e0, jnp.int32)
            rbase = jnp.full((16,), row0 + e0 // 4, jnp.int32)
            for f0h in range(2):
                vals = [
                    plsc.load_gather(in_v, [iota_f[f0h], gbase + zq[q]])
                    for q in range(16)
                ]
                for q in range(16):
                    t = zq[q]
                    row_idx = rbase + (t >> 2)
                    lane_idx = ((t & 3) << 5) + iota_f[f0h]
                    plsc.store_scatter(out_v, [row_idx, lane_idx], vals[q])
            return carry

        lax.fori_loop(0, 8, per_sub, 0)

    def per_batch(g, carry):
        c0 = start + g * TB
        pltpu.sync_copy(lut_t.at[:, pl.ds(c0 * 128, TB * 128)], in_v)

        def per_k(kk, carry2):
            permute_tile(kk, kk * 128, kk * 32)
            return carry2
        lax.fori_loop(0, TB, per_k, 0)
        pltpu.sync_copy(out_v, table.at[pl.ds(c0 * 32, TB * 32)])
        return carry

    lax.fori_loop(0, B0_FULL, per_batch, 0)

    def per_tile(t, carry):
        c = start + B0_FULL * TB + t
        is_last = c == NBLK - 1

        @pl.when(jnp.logical_not(is_last))
        def _():
            pltpu.sync_copy(lut_t.at[:, pl.ds(c * 128, 128)], in_v.at[:, pl.ds(0, 128)])

        @pl.when(is_last)
        def _():
            pltpu.sync_copy(tail, in_v.at[:, pl.ds(0, 128)])

        permute_tile(0, 0, 0)
        pltpu.sync_copy(out_v.at[pl.ds(0, 32)], table.at[pl.ds(c * 32, 32)])
        return carry

    lax.fori_loop(0, n_c - B0_FULL * TB, per_tile, 0)


def _relayout(lut):
    tail = jnp.pad(lut[(NBLK - 1) * 128:], ((0, VP - V), (0, 0))).T
    mesh = plsc.VectorSubcoreMesh(core_axis_name="c", subcore_axis_name="s")
    k = pl.kernel(
        _b0_body,
        mesh=mesh,
        out_type=jax.ShapeDtypeStruct((NBLK * 32, 128), jnp.float32),
        compiler_params=pltpu.CompilerParams(needs_layout_passes=False),
        scratch_types=[
            pltpu.VMEM((32, TB * 128), jnp.float32),
            pltpu.VMEM((TB * 32, 128), jnp.float32),
        ],
    )
    return k(lut.T, tail)


def _b1_body(idx_hbm, table, out5, idx_v, rows_v, trans_v, sem_g, sem_w):
    wid = lax.axis_index("s") * NC + lax.axis_index("c")
    blocks_per_w = NOUT // NW
    b0 = wid * blocks_per_w

    iota16 = lax.broadcasted_iota(jnp.int32, (16,), 0)

    def per_batch(g, carry):
        base_blk = b0 + g * K1
        pltpu.sync_copy(idx_hbm.at[pl.ds(base_blk * 128, K1 * 128)], idx_v)
        copies = [
            pltpu.async_copy(
                table.at[idx_v.at[pl.ds(k * 128, 128)]],
                rows_v.at[pl.ds(k * 128, 128)],
                sem_g,
            )
            for k in range(K1)
        ]
        for cp in copies:
            cp.wait()

        zq = [(iota16 + q) & 15 for q in range(16)]
        nvecs = [iota16 + 16 * t8 for t8 in range(8)]

        def per_block(k, carry2):
            blk = base_blk + k
            s = blk // 128
            nb = blk % 128
            kvec = jnp.full((16,), k, jnp.int32)
            for t8 in range(8):
                n0 = 16 * t8
                idx_n = k * 128 + n0 + iota16
                for f0h in range(2):
                    ts = [zq[q] + 16 * f0h for q in range(16)] if f0h else zq
                    vals = [
                        plsc.load_gather(rows_v, [idx_n, ts[q]])
                        for q in range(16)
                    ]
                    for q in range(16):
                        t = ts[q]
                        plsc.store_scatter(
                            trans_v, [kvec, t >> 3, t & 7, nvecs[t8]], vals[q]
                        )
            pltpu.async_copy(trans_v.at[k], out5.at[s, :, nb], sem_w)
            return carry2

        lax.fori_loop(0, K1, per_block, 0)
        for _ in range(K1):
            pltpu.make_async_copy(trans_v.at[0], out5.at[0, :, 0], sem_w).wait()
        return carry

    lax.fori_loop(0, blocks_per_w // K1, per_batch, 0)


def _lookup(idx_flat, table):
    mesh = plsc.VectorSubcoreMesh(core_axis_name="c", subcore_axis_name="s")
    k = pl.kernel(
        _b1_body,
        mesh=mesh,
        out_type=jax.ShapeDtypeStruct((S_TOK, 4, 128, 8, 128), jnp.float32),
        compiler_params=pltpu.CompilerParams(
            use_tc_tiling_on_sc=False, needs_layout_passes=False
        ),
        scratch_types=[
            pltpu.VMEM((K1 * 128,), jnp.int32),
            pltpu.VMEM((K1 * 128, D), jnp.float32),
            pltpu.VMEM((K1, 4, 8, 128), jnp.float32),
            pltpu.SemaphoreType.DMA,
            pltpu.SemaphoreType.DMA,
        ],
    )
    return k(idx_flat, table)


def kernel(token_ids, lut):
    idx_bt = token_ids.T.reshape(B).astype(jnp.int32)
    lin = _relayout(lut)
    table = lin.reshape(VP, D)
    out5 = _lookup(idx_bt, table)
    return jnp.transpose(out5, (2, 4, 0, 1, 3)).reshape(N_TOK, S_TOK, D)
